# baseline (speedup 1.0000x reference)
import functools

import jax
import jax.numpy as jnp
from jax import lax
from jax.experimental import pallas as pl
from jax.experimental.pallas import tpu as pltpu
from jax.experimental.pallas import tpu_sc as plsc

SEQ = 200
D = 64
BATCH = 4096
NC = 2
NS = 16
NW = NC * NS
BW = BATCH // NW
NBUF = 3
LANES = 16


def _sc_embed(xt, table, pe2):
    mesh = plsc.VectorSubcoreMesh(
        core_axis_name="c", subcore_axis_name="s",
        num_cores=NC, num_subcores=NS)

    @functools.partial(
        pl.kernel,
        out_type=jax.ShapeDtypeStruct((SEQ, D, BATCH), jnp.float32),
        mesh=mesh,
        scratch_types=[
            pltpu.VMEM((SEQ, D), jnp.float32),
            pltpu.VMEM((SEQ, BW), jnp.int32),
            pltpu.VMEM((NBUF, BW, D), jnp.float32),
            pltpu.VMEM((NBUF, D, BW), jnp.float32),
            pltpu.SemaphoreType.DMA((NBUF,)),
            pltpu.SemaphoreType.DMA((NBUF,)),
        ],
        compiler_params=pltpu.CompilerParams(use_tc_tiling_on_sc=False,
                                             needs_layout_passes=False),
    )
    def k(xt_hbm, tab_hbm, pe_hbm, out_hbm,
          pe_v, idx_v, gbuf, tbuf, s_gath, s_out):
        wid = lax.axis_index("s") * NC + lax.axis_index("c")
        b0 = wid * BW
        pltpu.sync_copy(pe_hbm, pe_v)
        pltpu.sync_copy(xt_hbm.at[:, pl.ds(b0, BW)], idx_v)

        row_ids = [jnp.arange(LANES, dtype=jnp.int32) + bg * LANES
                   for bg in range(BW // LANES)]

        def start_gather(s):
            b = lax.rem(s, NBUF)
            pltpu.async_copy(tab_hbm.at[idx_v.at[s]], gbuf.at[b],
                             s_gath.at[b])

        def transpose_store(s):
            b = lax.rem(s, NBUF)
            pltpu.make_async_copy(
                tab_hbm.at[idx_v.at[0]], gbuf.at[b], s_gath.at[b]).wait()

            srow = jnp.full((LANES,), s, dtype=jnp.int32)

            def d_body(d, carry):
                col = jnp.full((LANES,), d, dtype=jnp.int32)
                pev = plsc.load_gather(pe_v, [srow, col])
                for bg in range(BW // LANES):
                    v = plsc.load_gather(gbuf.at[b], [row_ids[bg], col])
                    tbuf[b, d, pl.ds(bg * LANES, LANES)] = v + pev
                return carry

            lax.fori_loop(0, D, d_body, 0)
            pltpu.async_copy(tbuf.at[b], out_hbm.at[s, :, pl.ds(b0, BW)],
                             s_out.at[b])

        def wait_store(s):
            b = lax.rem(s, NBUF)
            pltpu.make_async_copy(
                tbuf.at[b], out_hbm.at[0, :, pl.ds(b0, BW)],
                s_out.at[b]).wait()

        def body(i, carry):
            @pl.when(i < SEQ)
            def _():
                @pl.when(i >= NBUF)
                def _():
                    wait_store(i - NBUF)
                start_gather(i)

            @pl.when(jnp.logical_and(i >= 1, i - 1 < SEQ))
            def _():
                transpose_store(i - 1)
            return carry

        lax.fori_loop(0, SEQ + 1, body, 0)
        for j in range(min(NBUF, SEQ)):
            wait_store(SEQ - 1 - j)

    return k(xt, table, pe2)


def kernel(x, table, pe):
    xt = x.astype(jnp.int32).T
    pe2 = pe.reshape(SEQ, D)
    out = _sc_embed(xt, table, pe2)
    return out.transpose(2, 0, 1)


# speedup vs baseline: 1.9120x; 1.9120x over previous
import functools

import jax
import jax.numpy as jnp
from jax import lax
from jax.experimental import pallas as pl
from jax.experimental.pallas import tpu as pltpu
from jax.experimental.pallas import tpu_sc as plsc

SEQ = 200
D = 64
BATCH = 4096
NC = 2
NS = 16
NW = NC * NS
BW = BATCH // NW
BWP = BW + 1
NBUF = 3
LANES = 16


def _sc_embed(xt, table, pe2):
    mesh = plsc.VectorSubcoreMesh(
        core_axis_name="c", subcore_axis_name="s",
        num_cores=NC, num_subcores=NS)

    @functools.partial(
        pl.kernel,
        out_type=jax.ShapeDtypeStruct((SEQ, D, BATCH), jnp.float32),
        mesh=mesh,
        scratch_types=[
            pltpu.VMEM((SEQ, D), jnp.float32),
            pltpu.VMEM((SEQ, BW), jnp.int32),
            pltpu.VMEM((NBUF, BW, D), jnp.float32),
            pltpu.VMEM((NBUF, D, BWP), jnp.float32),
            pltpu.SemaphoreType.DMA((NBUF,)),
            pltpu.SemaphoreType.DMA((NBUF,)),
        ],
        compiler_params=pltpu.CompilerParams(use_tc_tiling_on_sc=False,
                                             needs_layout_passes=False),
    )
    def k(xt_hbm, tab_hbm, pe_hbm, out_hbm,
          pe_v, idx_v, gbuf, tbuf, s_gath, s_out):
        wid = lax.axis_index("s") * NC + lax.axis_index("c")
        b0 = wid * BW
        pltpu.sync_copy(pe_hbm, pe_v)
        pltpu.sync_copy(xt_hbm.at[:, pl.ds(b0, BW)], idx_v)

        dvecs = [jnp.arange(LANES, dtype=jnp.int32) + l * LANES
                 for l in range(D // LANES)]

        def start_gather(s):
            b = lax.rem(s, NBUF)
            pltpu.async_copy(tab_hbm.at[idx_v.at[s]], gbuf.at[b],
                             s_gath.at[b])

        def transpose_store(s):
            b = lax.rem(s, NBUF)
            pltpu.make_async_copy(
                tab_hbm.at[idx_v.at[0]], gbuf.at[b], s_gath.at[b]).wait()
            pvs = [pe_v[s, pl.ds(l * LANES, LANES)]
                   for l in range(D // LANES)]
            tb = tbuf.at[b]

            def j_body(j, carry):
                jv = jnp.full((LANES,), j, dtype=jnp.int32)
                for l in range(D // LANES):
                    v = gbuf[b, j, pl.ds(l * LANES, LANES)] + pvs[l]
                    plsc.store_scatter(tb, [dvecs[l], jv], v)
                return carry

            lax.fori_loop(0, BW, j_body, 0)
            pltpu.async_copy(tbuf.at[b, :, pl.ds(0, BW)],
                             out_hbm.at[s, :, pl.ds(b0, BW)],
                             s_out.at[b])

        def wait_store(s):
            b = lax.rem(s, NBUF)
            pltpu.make_async_copy(
                tbuf.at[b, :, pl.ds(0, BW)],
                out_hbm.at[0, :, pl.ds(b0, BW)],
                s_out.at[b]).wait()

        def body(i, carry):
            @pl.when(i < SEQ)
            def _():
                @pl.when(i >= NBUF)
                def _():
                    wait_store(i - NBUF)
                start_gather(i)

            @pl.when(jnp.logical_and(i >= 1, i - 1 < SEQ))
            def _():
                transpose_store(i - 1)
            return carry

        lax.fori_loop(0, SEQ + 1, body, 0)
        for j in range(min(NBUF, SEQ)):
            wait_store(SEQ - 1 - j)

    return k(xt, table, pe2)


def kernel(x, table, pe):
    xt = x.astype(jnp.int32).T
    pe2 = pe.reshape(SEQ, D)
    out = _sc_embed(xt, table, pe2)
    return out.transpose(2, 0, 1)


# speedup vs baseline: 3.0826x; 1.6122x over previous
import functools

import jax
import jax.numpy as jnp
from jax import lax
from jax.experimental import pallas as pl
from jax.experimental.pallas import tpu as pltpu
from jax.experimental.pallas import tpu_sc as plsc

SEQ = 200
D = 64
BATCH = 4096
NC = 2
NS = 16
NW = NC * NS
BW = BATCH // NW
BWP = BW + 1
NBUF = 3
LANES = 16


def _sc_embed(xt, table, pe2):
    mesh = plsc.VectorSubcoreMesh(
        core_axis_name="c", subcore_axis_name="s",
        num_cores=NC, num_subcores=NS)

    @functools.partial(
        pl.kernel,
        out_type=jax.ShapeDtypeStruct((SEQ, D, BATCH), jnp.float32),
        mesh=mesh,
        scratch_types=[
            pltpu.VMEM((SEQ, D), jnp.float32),
            pltpu.VMEM((SEQ, BW), jnp.int32),
            pltpu.VMEM((NBUF, BW, D), jnp.float32),
            pltpu.VMEM((NBUF, D, BWP), jnp.float32),
            pltpu.SemaphoreType.DMA((NBUF,)),
            pltpu.SemaphoreType.DMA((NBUF,)),
        ],
        compiler_params=pltpu.CompilerParams(use_tc_tiling_on_sc=False,
                                             needs_layout_passes=False),
    )
    def k(xt_hbm, tab_hbm, pe_hbm, out_hbm,
          pe_v, idx_v, gbuf, tbuf, s_gath, s_out):
        wid = lax.axis_index("s") * NC + lax.axis_index("c")
        b0 = wid * BW
        pltpu.sync_copy(pe_hbm, pe_v)
        pltpu.sync_copy(xt_hbm.at[:, pl.ds(b0, BW)], idx_v)

        dvecs = [jnp.arange(LANES, dtype=jnp.int32) + l * LANES
                 for l in range(D // LANES)]

        def start_gather(s):
            b = lax.rem(s, NBUF)
            pltpu.async_copy(tab_hbm.at[idx_v.at[s]], gbuf.at[b],
                             s_gath.at[b])

        def transpose_store(s):
            b = lax.rem(s, NBUF)
            pltpu.make_async_copy(
                tab_hbm.at[idx_v.at[0]], gbuf.at[b], s_gath.at[b]).wait()
            pvs = [pe_v[s, pl.ds(l * LANES, LANES)]
                   for l in range(D // LANES)]
            tb = tbuf.at[b]

            @plsc.parallel_loop(0, BW, unroll=8)
            def j_body(j):
                jv = jnp.full((LANES,), j, dtype=jnp.int32)
                for l in range(D // LANES):
                    v = gbuf[b, j, pl.ds(l * LANES, LANES)] + pvs[l]
                    plsc.store_scatter(tb, [dvecs[l], jv], v)
            pltpu.async_copy(tbuf.at[b, :, pl.ds(0, BW)],
                             out_hbm.at[s, :, pl.ds(b0, BW)],
                             s_out.at[b])

        def wait_store(s):
            b = lax.rem(s, NBUF)
            pltpu.make_async_copy(
                tbuf.at[b, :, pl.ds(0, BW)],
                out_hbm.at[0, :, pl.ds(b0, BW)],
                s_out.at[b]).wait()

        def body(i, carry):
            @pl.when(i < SEQ)
            def _():
                @pl.when(i >= NBUF)
                def _():
                    wait_store(i - NBUF)
                start_gather(i)

            @pl.when(jnp.logical_and(i >= 1, i - 1 < SEQ))
            def _():
                transpose_store(i - 1)
            return carry

        lax.fori_loop(0, SEQ + 1, body, 0)
        for j in range(min(NBUF, SEQ)):
            wait_store(SEQ - 1 - j)

    return k(xt, table, pe2)


def kernel(x, table, pe):
    xt = x.astype(jnp.int32).T
    pe2 = pe.reshape(SEQ, D)
    out = _sc_embed(xt, table, pe2)
    return out.transpose(2, 0, 1)


# speedup vs baseline: 3.3491x; 1.0865x over previous
import functools

import jax
import jax.numpy as jnp
from jax import lax
from jax.experimental import pallas as pl
from jax.experimental.pallas import tpu as pltpu
from jax.experimental.pallas import tpu_sc as plsc

SEQ = 200
D = 64
BATCH = 4096
NC = 2
NS = 16
NW = NC * NS
BW = BATCH // NW
BWP = BW + 1
NBUF = 3
LANES = 16


def _sc_embed(xt, table, pe2):
    mesh = plsc.VectorSubcoreMesh(
        core_axis_name="c", subcore_axis_name="s",
        num_cores=NC, num_subcores=NS)

    @functools.partial(
        pl.kernel,
        out_type=jax.ShapeDtypeStruct((SEQ, D, BATCH), jnp.float32),
        mesh=mesh,
        scratch_types=[
            pltpu.VMEM((SEQ, D), jnp.float32),
            pltpu.VMEM((SEQ, BW), jnp.int32),
            pltpu.VMEM((NBUF, BW, D), jnp.float32),
            pltpu.VMEM((NBUF, D, BWP), jnp.float32),
            pltpu.SemaphoreType.DMA((NBUF,)),
            pltpu.SemaphoreType.DMA((NBUF,)),
        ],
        compiler_params=pltpu.CompilerParams(use_tc_tiling_on_sc=False,
                                             needs_layout_passes=False),
    )
    def k(xt_hbm, tab_hbm, pe_hbm, out_hbm,
          pe_v, idx_v, gbuf, tbuf, s_gath, s_out):
        wid = lax.axis_index("s") * NC + lax.axis_index("c")
        b0 = wid * BW
        pltpu.sync_copy(pe_hbm, pe_v)
        pltpu.sync_copy(xt_hbm.at[:, pl.ds(b0, BW)], idx_v)

        dvecs = [jnp.arange(LANES, dtype=jnp.int32) + l * LANES
                 for l in range(D // LANES)]

        def start_gather(s):
            b = lax.rem(s, NBUF)
            pltpu.async_copy(tab_hbm.at[idx_v.at[s]], gbuf.at[b],
                             s_gath.at[b])

        def transpose_store(s):
            b = lax.rem(s, NBUF)
            pltpu.make_async_copy(
                tab_hbm.at[idx_v.at[0]], gbuf.at[b], s_gath.at[b]).wait()
            pvs = [pe_v[s, pl.ds(l * LANES, LANES)]
                   for l in range(D // LANES)]
            tb = tbuf.at[b]

            @plsc.parallel_loop(0, BW, unroll=16)
            def j_body(j):
                jv = jnp.full((LANES,), j, dtype=jnp.int32)
                for l in range(D // LANES):
                    v = gbuf[b, j, pl.ds(l * LANES, LANES)] + pvs[l]
                    plsc.store_scatter(tb, [dvecs[l], jv], v)
            pltpu.async_copy(tbuf.at[b, :, pl.ds(0, BW)],
                             out_hbm.at[s, :, pl.ds(b0, BW)],
                             s_out.at[b])

        def wait_store(s):
            b = lax.rem(s, NBUF)
            pltpu.make_async_copy(
                tbuf.at[b, :, pl.ds(0, BW)],
                out_hbm.at[0, :, pl.ds(b0, BW)],
                s_out.at[b]).wait()

        def body(i, carry):
            @pl.when(i < SEQ)
            def _():
                @pl.when(i >= NBUF)
                def _():
                    wait_store(i - NBUF)
                start_gather(i)

            @pl.when(jnp.logical_and(i >= 1, i - 1 < SEQ))
            def _():
                transpose_store(i - 1)
            return carry

        lax.fori_loop(0, SEQ + 1, body, 0)
        for j in range(min(NBUF, SEQ)):
            wait_store(SEQ - 1 - j)

    return k(xt, table, pe2)


def kernel(x, table, pe):
    xt = x.astype(jnp.int32).T
    pe2 = pe.reshape(SEQ, D)
    out = _sc_embed(xt, table, pe2)
    return out.transpose(2, 0, 1)
